# Initial kernel scaffold; baseline (speedup 1.0000x reference)
import jax
import jax.numpy as jnp
from jax.experimental import pallas as pl


def kernel(node_features, edge_index, edge_type, rgcn_weight, rgcn_root, rgcn_bias, Wq, bq, Wk, bk, Wv, bv, Wskip, bskip, bn_gamma, bn_beta):
    raise NotImplementedError("write your pallas kernel here")


# speedup vs baseline: 1.8493x; 1.8493x over previous
import jax
import jax.numpy as jnp
from jax.experimental import pallas as pl


def _mm_kernel(x_ref, w_ref, b_ref, o_ref):
    o_ref[...] = (
        jnp.dot(x_ref[...], w_ref[...], preferred_element_type=jnp.float32)
        + b_ref[...]
    )


def _matmul_bias(x, w, b, bm, bn):
    M, K = x.shape
    Nn = w.shape[1]
    grid = (M // bm, Nn // bn)
    return pl.pallas_call(
        _mm_kernel,
        grid=grid,
        in_specs=[
            pl.BlockSpec((bm, K), lambda i, j: (i, 0)),
            pl.BlockSpec((K, bn), lambda i, j: (0, j)),
            pl.BlockSpec((1, bn), lambda i, j: (0, j)),
        ],
        out_specs=pl.BlockSpec((bm, bn), lambda i, j: (i, j)),
        out_shape=jax.ShapeDtypeStruct((M, Nn), jnp.float32),
    )(x, w, b.reshape(1, -1))


def _bnstats_kernel(x_ref, s_ref, q_ref):
    @pl.when(pl.program_id(0) == 0)
    def _init():
        s_ref[...] = jnp.zeros_like(s_ref)
        q_ref[...] = jnp.zeros_like(q_ref)

    xb = x_ref[...]
    s_ref[...] = s_ref[...] + jnp.sum(xb, axis=0, keepdims=True)
    q_ref[...] = q_ref[...] + jnp.sum(xb * xb, axis=0, keepdims=True)


def _bn_stats(x, bm):
    M, Nn = x.shape
    return pl.pallas_call(
        _bnstats_kernel,
        grid=(M // bm,),
        in_specs=[pl.BlockSpec((bm, Nn), lambda i: (i, 0))],
        out_specs=[
            pl.BlockSpec((1, Nn), lambda i: (0, 0)),
            pl.BlockSpec((1, Nn), lambda i: (0, 0)),
        ],
        out_shape=[
            jax.ShapeDtypeStruct((1, Nn), jnp.float32),
            jax.ShapeDtypeStruct((1, Nn), jnp.float32),
        ],
    )(x)


def _bnapply_kernel(x_ref, sc_ref, sh_ref, o_ref):
    y = x_ref[...] * sc_ref[...] + sh_ref[...]
    o_ref[...] = jnp.where(y >= 0.0, y, 0.01 * y)


def _bn_apply(x, scale, shift, bm):
    M, Nn = x.shape
    return pl.pallas_call(
        _bnapply_kernel,
        grid=(M // bm,),
        in_specs=[
            pl.BlockSpec((bm, Nn), lambda i: (i, 0)),
            pl.BlockSpec((1, Nn), lambda i: (0, 0)),
            pl.BlockSpec((1, Nn), lambda i: (0, 0)),
        ],
        out_specs=pl.BlockSpec((bm, Nn), lambda i: (i, 0)),
        out_shape=jax.ShapeDtypeStruct((M, Nn), jnp.float32),
    )(x, scale.reshape(1, -1), shift.reshape(1, -1))


def kernel(node_features, edge_index, edge_type, rgcn_weight, rgcn_root,
           rgcn_bias, Wq, bq, Wk, bk, Wv, bv, Wskip, bskip, bn_gamma, bn_beta):
    x = node_features
    N, G = x.shape
    R = rgcn_weight.shape[0]
    H1 = rgcn_root.shape[1]
    src = edge_index[0]
    dst = edge_index[1]
    E = src.shape[0]

    sid = edge_type * N + dst
    ones = jnp.ones((E,), jnp.float32)
    cnt = jax.ops.segment_sum(ones, sid, num_segments=R * N)
    xs = jnp.take(x, src, axis=0)
    S = jax.ops.segment_sum(xs, sid, num_segments=R * N)
    meanr = S / jnp.maximum(cnt, 1.0)[:, None]
    meanr = meanr.reshape(R, N, G).transpose(1, 0, 2).reshape(N, R * G)
    Xcat = jnp.concatenate([x, meanr], axis=1)
    Wcat = jnp.concatenate(
        [rgcn_root, rgcn_weight.reshape(R * G, H1)], axis=0)
    x1 = _matmul_bias(Xcat, Wcat, rgcn_bias, bm=400, bn=512)

    HC = Wq.shape[1]
    H = 4
    C = HC // H
    W2 = jnp.concatenate([Wq, Wk, Wv, Wskip], axis=1)
    b2 = jnp.concatenate([bq, bk, bv, bskip], axis=0)
    qkvs = _matmul_bias(x1, W2, b2, bm=400, bn=2048)
    q = qkvs[:, :HC].reshape(N, H, C)
    k = qkvs[:, HC:2 * HC].reshape(N, H, C)
    v = qkvs[:, 2 * HC:3 * HC].reshape(N, H, C)
    skipt = qkvs[:, 3 * HC:]

    alpha = jnp.sum(jnp.take(q, dst, axis=0) * jnp.take(k, src, axis=0),
                    axis=-1) / jnp.sqrt(float(C))
    amax = jax.ops.segment_max(alpha, dst, num_segments=N)
    amax = jnp.where(jnp.isfinite(amax), amax, 0.0)
    ex = jnp.exp(alpha - jnp.take(amax, dst, axis=0))
    denom = jax.ops.segment_sum(ex, dst, num_segments=N)
    attn = ex / jnp.maximum(jnp.take(denom, dst, axis=0), 1e-16)
    msg = (attn[:, :, None] * jnp.take(v, src, axis=0)).reshape(E, HC)
    agg = jax.ops.segment_sum(msg, dst, num_segments=N)
    out_raw = agg + skipt

    ssum, sqsum = _bn_stats(out_raw, bm=400)
    mean = (ssum[0] / N)
    var = sqsum[0] / N - mean * mean
    inv = bn_gamma / jnp.sqrt(var + 1e-5)
    shift = bn_beta - mean * inv
    return _bn_apply(out_raw, inv, shift, bm=400)
